# baseline (speedup 1.0000x reference)
import functools

import jax
import jax.numpy as jnp
from jax import lax
from jax.experimental import pallas as pl
from jax.experimental.pallas import tpu as pltpu
from jax.experimental.pallas import tpu_sc as plsc

NC = 2
NS = 16
NW = NC * NS
LANES = 16

HID = 128
POS_VOCAB = 13
TOTAL = 1024 * 200
NROWS = TOTAL // NW
CHUNK = 128
NCHUNK = NROWS // CHUNK
NBUF = 5
LOOKAHEAD = 4


def _emb_kernel(ids_hbm, pids_hbm, word_hbm, pos_hbm, out_hbm,
                idx_v, pidx_v, pos_local, wbufs, gsems, osems):
    wid = lax.axis_index("s") * NC + lax.axis_index("c")

    pltpu.sync_copy(ids_hbm.at[wid], idx_v)
    pltpu.sync_copy(pids_hbm.at[wid], pidx_v)
    pltpu.sync_copy(pos_hbm, pos_local)

    for b in range(LOOKAHEAD):
        pltpu.async_copy(word_hbm.at[idx_v.at[b]], wbufs[b], gsems[b])

    iota = lax.iota(jnp.int32, LANES)
    row_ids = [g * LANES + iota for g in range(CHUNK // LANES)]

    @pl.loop(0, NCHUNK, step=NBUF)
    def outer(j0):
        for b in range(NBUF):
            j = j0 + b
            pltpu.make_async_copy(word_hbm.at[idx_v.at[j]], wbufs[b],
                                  gsems[b]).wait()

            @plsc.parallel_loop(0, CHUNK // LANES, unroll=2)
            def group(g):
                pv = pidx_v[j, pl.ds(g * LANES, LANES)]
                for l in range(LANES):
                    p = pv[l]
                    r = g * LANES + l
                    for c in range(HID // LANES):
                        x = pos_local[p, pl.ds(c * LANES, LANES)]
                        plsc.addupdate(
                            wbufs[b].at[r, pl.ds(c * LANES, LANES)], x)

            pltpu.async_copy(wbufs[b], out_hbm.at[wid, j], osems[b])

            nb = (b + LOOKAHEAD) % NBUF

            @pl.when(j + LOOKAHEAD < NCHUNK)
            def _prefetch():
                @pl.when(j + LOOKAHEAD >= NBUF)
                def _drain():
                    pltpu.make_async_copy(wbufs[nb], out_hbm.at[wid, 0],
                                          osems[nb]).wait()

                pltpu.async_copy(word_hbm.at[idx_v.at[j + LOOKAHEAD]],
                                 wbufs[nb], gsems[nb])

    for b in range(NBUF):
        pltpu.make_async_copy(wbufs[b], out_hbm.at[wid, 0], osems[b]).wait()


@jax.jit
def _run(ids3, pids3, word_embeddings, pos_embeddings):
    mesh = plsc.VectorSubcoreMesh(core_axis_name="c", subcore_axis_name="s")
    k = functools.partial(
        pl.kernel,
        out_type=jax.ShapeDtypeStruct((NW, NCHUNK, CHUNK, HID), jnp.float32),
        mesh=mesh,
        compiler_params=pltpu.CompilerParams(needs_layout_passes=False),
        scratch_types=[
            pltpu.VMEM((NCHUNK, CHUNK), jnp.int32),
            pltpu.VMEM((NCHUNK, CHUNK), jnp.int32),
            pltpu.VMEM((POS_VOCAB, HID), jnp.float32),
            [pltpu.VMEM((CHUNK, HID), jnp.float32) for _ in range(NBUF)],
            [pltpu.SemaphoreType.DMA for _ in range(NBUF)],
            [pltpu.SemaphoreType.DMA for _ in range(NBUF)],
        ],
    )(_emb_kernel)
    return k(ids3, pids3, word_embeddings, pos_embeddings)


def kernel(input_ids, pos_ids, word_embeddings, position_embeddings,
           token_type_embeddings, pos_embeddings):
    del position_embeddings, token_type_embeddings
    B, S = input_ids.shape
    ids3 = input_ids.reshape(NW, NCHUNK, CHUNK).astype(jnp.int32)
    pids3 = pos_ids.reshape(NW, NCHUNK, CHUNK).astype(jnp.int32)
    out = _run(ids3, pids3, word_embeddings, pos_embeddings)
    return out.reshape(B, S, HID)


# speedup vs baseline: 1.4982x; 1.4982x over previous
import functools

import jax
import jax.numpy as jnp
from jax import lax
from jax.experimental import pallas as pl
from jax.experimental.pallas import tpu as pltpu
from jax.experimental.pallas import tpu_sc as plsc

NC = 2
NS = 16
NW = NC * NS
LANES = 16

HID = 128
POS_VOCAB = 13
TOTAL = 1024 * 200
NROWS = TOTAL // NW
CHUNK = 64
NCHUNK = NROWS // CHUNK
NBUF = 12
LOOKAHEAD = 6


def _emb_kernel(ids_hbm, pids_hbm, word_hbm, pos_hbm, out_hbm,
                idx_v, pidx_v, pos_local, wbuf, gsem, osem):
    wid = lax.axis_index("s") * NC + lax.axis_index("c")

    pltpu.sync_copy(ids_hbm.at[wid], idx_v)
    pltpu.sync_copy(pids_hbm.at[wid], pidx_v)
    pltpu.sync_copy(pos_hbm, pos_local)

    for b in range(LOOKAHEAD):
        pltpu.async_copy(word_hbm.at[idx_v.at[b]], wbuf.at[b], gsem.at[b])

    @pl.loop(0, NCHUNK)
    def chunk(j):
        slot = lax.rem(j, NBUF)
        pltpu.make_async_copy(word_hbm.at[idx_v.at[j]], wbuf.at[slot],
                              gsem.at[slot]).wait()

        @plsc.parallel_loop(0, CHUNK // LANES)
        def group(g):
            pv = pidx_v[j, pl.ds(g * LANES, LANES)]
            for l in range(LANES):
                p = pv[l]
                r = g * LANES + l
                for c in range(HID // LANES):
                    x = pos_local[p, pl.ds(c * LANES, LANES)]
                    plsc.addupdate(
                        wbuf.at[slot, r, pl.ds(c * LANES, LANES)], x)

        pltpu.async_copy(wbuf.at[slot], out_hbm.at[wid, j], osem.at[slot])

        nj = j + LOOKAHEAD
        nslot = lax.rem(nj, NBUF)

        @pl.when(nj < NCHUNK)
        def _prefetch():
            @pl.when(nj >= NBUF)
            def _drain():
                pltpu.make_async_copy(wbuf.at[nslot], out_hbm.at[wid, 0],
                                      osem.at[nslot]).wait()

            pltpu.async_copy(word_hbm.at[idx_v.at[nj]], wbuf.at[nslot],
                             gsem.at[nslot])

    @pl.loop(NCHUNK - NBUF, NCHUNK)
    def drain(j):
        slot = lax.rem(j, NBUF)
        pltpu.make_async_copy(wbuf.at[slot], out_hbm.at[wid, 0],
                              osem.at[slot]).wait()


@jax.jit
def _run(ids3, pids3, word_embeddings, pos_embeddings):
    mesh = plsc.VectorSubcoreMesh(core_axis_name="c", subcore_axis_name="s")
    k = functools.partial(
        pl.kernel,
        out_type=jax.ShapeDtypeStruct((NW, NCHUNK, CHUNK, HID), jnp.float32),
        mesh=mesh,
        compiler_params=pltpu.CompilerParams(needs_layout_passes=False),
        scratch_types=[
            pltpu.VMEM((NCHUNK, CHUNK), jnp.int32),
            pltpu.VMEM((NCHUNK, CHUNK), jnp.int32),
            pltpu.VMEM((POS_VOCAB, HID), jnp.float32),
            pltpu.VMEM((NBUF, CHUNK, HID), jnp.float32),
            pltpu.SemaphoreType.DMA((NBUF,)),
            pltpu.SemaphoreType.DMA((NBUF,)),
        ],
    )(_emb_kernel)
    return k(ids3, pids3, word_embeddings, pos_embeddings)


def kernel(input_ids, pos_ids, word_embeddings, position_embeddings,
           token_type_embeddings, pos_embeddings):
    del position_embeddings, token_type_embeddings
    B, S = input_ids.shape
    ids3 = input_ids.reshape(NW, NCHUNK, CHUNK).astype(jnp.int32)
    pids3 = pos_ids.reshape(NW, NCHUNK, CHUNK).astype(jnp.int32)
    out = _run(ids3, pids3, word_embeddings, pos_embeddings)
    return out.reshape(B, S, HID)


# speedup vs baseline: 2.0453x; 1.3652x over previous
import functools

import jax
import jax.numpy as jnp
from jax import lax
from jax.experimental import pallas as pl
from jax.experimental.pallas import tpu as pltpu
from jax.experimental.pallas import tpu_sc as plsc

NC = 2
NS = 16
NW = NC * NS
LANES = 16

HID = 128
POS_VOCAB = 13
TOTAL = 1024 * 200
NROWS = TOTAL // NW
CHUNK = 128
NCHUNK = NROWS // CHUNK
NBUF = 6
LOOKAHEAD = 3


def _emb_kernel(ids_hbm, pids_hbm, word_hbm, pos_hbm, out_hbm,
                idx_v, pidx_v, pos_local, wbuf, gsem, osem):
    wid = lax.axis_index("s") * NC + lax.axis_index("c")

    pltpu.sync_copy(ids_hbm.at[wid], idx_v)
    pltpu.sync_copy(pids_hbm.at[wid], pidx_v)
    pltpu.sync_copy(pos_hbm, pos_local)

    for b in range(LOOKAHEAD):
        pltpu.async_copy(word_hbm.at[idx_v.at[b]], wbuf.at[b], gsem.at[b])

    @pl.loop(0, NCHUNK)
    def chunk(j):
        slot = lax.rem(j, NBUF)
        pltpu.make_async_copy(word_hbm.at[idx_v.at[j]], wbuf.at[slot],
                              gsem.at[slot]).wait()

        @plsc.parallel_loop(0, CHUNK // LANES)
        def group(g):
            pv = pidx_v[j, pl.ds(g * LANES, LANES)]
            for l in range(LANES):
                p = pv[l]
                r = g * LANES + l
                for c in range(HID // LANES):
                    x = pos_local[p, pl.ds(c * LANES, LANES)]
                    plsc.addupdate(
                        wbuf.at[slot, r, pl.ds(c * LANES, LANES)], x)

        pltpu.async_copy(wbuf.at[slot], out_hbm.at[wid, j], osem.at[slot])

        nj = j + LOOKAHEAD
        nslot = lax.rem(nj, NBUF)

        @pl.when(nj < NCHUNK)
        def _prefetch():
            @pl.when(nj >= NBUF)
            def _drain():
                pltpu.make_async_copy(wbuf.at[nslot], out_hbm.at[wid, 0],
                                      osem.at[nslot]).wait()

            pltpu.async_copy(word_hbm.at[idx_v.at[nj]], wbuf.at[nslot],
                             gsem.at[nslot])

    @pl.loop(NCHUNK - NBUF, NCHUNK)
    def drain(j):
        slot = lax.rem(j, NBUF)
        pltpu.make_async_copy(wbuf.at[slot], out_hbm.at[wid, 0],
                              osem.at[slot]).wait()


@jax.jit
def _run(ids3, pids3, word_embeddings, pos_embeddings):
    mesh = plsc.VectorSubcoreMesh(core_axis_name="c", subcore_axis_name="s")
    k = functools.partial(
        pl.kernel,
        out_type=jax.ShapeDtypeStruct((NW, NCHUNK, CHUNK, HID), jnp.float32),
        mesh=mesh,
        compiler_params=pltpu.CompilerParams(needs_layout_passes=False),
        scratch_types=[
            pltpu.VMEM((NCHUNK, CHUNK), jnp.int32),
            pltpu.VMEM((NCHUNK, CHUNK), jnp.int32),
            pltpu.VMEM((POS_VOCAB, HID), jnp.float32),
            pltpu.VMEM((NBUF, CHUNK, HID), jnp.float32),
            pltpu.SemaphoreType.DMA((NBUF,)),
            pltpu.SemaphoreType.DMA((NBUF,)),
        ],
    )(_emb_kernel)
    return k(ids3, pids3, word_embeddings, pos_embeddings)


def kernel(input_ids, pos_ids, word_embeddings, position_embeddings,
           token_type_embeddings, pos_embeddings):
    del position_embeddings, token_type_embeddings
    B, S = input_ids.shape
    ids3 = input_ids.reshape(NW, NCHUNK, CHUNK).astype(jnp.int32)
    pids3 = pos_ids.reshape(NW, NCHUNK, CHUNK).astype(jnp.int32)
    out = _run(ids3, pids3, word_embeddings, pos_embeddings)
    return out.reshape(B, S, HID)
